# baseline (speedup 1.0000x reference)
import functools

import jax
import jax.numpy as jnp
from jax import lax
from jax.experimental import pallas as pl
from jax.experimental.pallas import tpu as pltpu, tpu_sc as plsc

_G = 20
_I = 26
_O = 64


def _prep_body(sw_ref, scale_ref, swf_ref):
    swf_ref[...] = sw_ref[...] * scale_ref[...][:, :, None]


def _fold_scale(spline_weights, scale):
    I, O, G = spline_weights.shape
    return pl.pallas_call(
        _prep_body,
        in_specs=[pl.BlockSpec((I, O, G), lambda: (0, 0, 0)),
                  pl.BlockSpec((I, O), lambda: (0, 0))],
        out_specs=pl.BlockSpec((I, O, G), lambda: (0, 0, 0)),
        out_shape=jax.ShapeDtypeStruct((I, O, G), jnp.float32),
    )(spline_weights, scale)


def _sc_body(rows, nchunk, x_hbm, swf_hbm, aux_hbm, out_hbm,
             xs_v, sw_v, aux_v, base_s, t_s, out_v):
    G, I, O = _G, _I, _O
    info = plsc.get_sparse_core_info()
    nc = info.num_cores
    wid = lax.axis_index("s") * nc + lax.axis_index("c")
    base_row = wid * rows

    pltpu.sync_copy(x_hbm.at[pl.ds(base_row * I, rows * I)], xs_v)
    pltpu.sync_copy(swf_hbm, sw_v)
    pltpu.sync_copy(aux_hbm, aux_v)

    lanes = lax.iota(jnp.int32, 16)
    g0v = aux_v[0:16]
    gLv = aux_v[16:32]
    hv = (gLv - g0v) / (G - 1.0)
    inv_hv = (G - 1.0) / (gLv - g0v)
    invtv = 1.0 / (hv + 1e-08)

    def chunk_body(c, carry):
        row_vec = c * 16 + lanes
        for i in range(I):
            xv = plsc.load_gather(xs_v, [row_vec * I + i])
            xc = jnp.minimum(jnp.maximum(xv, g0v), gLv)
            ji = jnp.clip(((xc - g0v) * inv_hv).astype(jnp.int32), 0, G - 2)
            tv = (xc - (g0v + ji.astype(jnp.float32) * hv)) * invtv
            base_s[i * 16:(i + 1) * 16] = ji + (i * O * G)
            t_s[i * 16:(i + 1) * 16] = tv

        def og_body(og, carry2):
            o20 = og * (16 * G)
            acc = [jnp.zeros((16,), jnp.float32) for _ in range(16)]
            for i in range(I):
                bv = base_s[i * 16:(i + 1) * 16] + o20
                tv = t_s[i * 16:(i + 1) * 16]
                for k in range(16):
                    a0 = bv + (k * G)
                    y0 = plsc.load_gather(sw_v, [a0])
                    y1 = plsc.load_gather(sw_v, [a0 + 1])
                    acc[k] = acc[k] + (y0 + tv * (y1 - y0))
            ob = row_vec * O + og * 16
            for k in range(16):
                plsc.store_scatter(out_v, [ob + k], acc[k])
            return carry2

        lax.fori_loop(0, O // 16, og_body, 0, unroll=False)
        return carry

    lax.fori_loop(0, nchunk, chunk_body, 0, unroll=False)
    pltpu.sync_copy(out_v, out_hbm.at[pl.ds(base_row * O, rows * O)])


def sc_kan(x, grid_points, spline_weights, scale):
    B, I = x.shape
    G = grid_points.shape[0]
    O = spline_weights.shape[1]
    info = plsc.get_sparse_core_info()
    nw = info.num_cores * info.num_subcores
    rows = B // nw
    nchunk = rows // 16

    swf = _fold_scale(spline_weights, scale).reshape(-1)
    aux = jnp.concatenate(
        [jnp.full((16,), grid_points[0], jnp.float32),
         jnp.full((16,), grid_points[G - 1], jnp.float32),
         jnp.zeros((96,), jnp.float32)])

    mesh = plsc.VectorSubcoreMesh(core_axis_name="c", subcore_axis_name="s")
    body = functools.partial(_sc_body, rows, nchunk)
    out_flat = pl.kernel(
        body,
        out_type=jax.ShapeDtypeStruct((B * O,), jnp.float32),
        mesh=mesh,
        compiler_params=pltpu.CompilerParams(needs_layout_passes=False),
        scratch_types=[
            pltpu.VMEM((rows * I,), jnp.float32),
            pltpu.VMEM((I * O * G,), jnp.float32),
            pltpu.VMEM((128,), jnp.float32),
            pltpu.VMEM((I * 16,), jnp.int32),
            pltpu.VMEM((I * 16,), jnp.float32),
            pltpu.VMEM((rows * O,), jnp.float32),
        ],
    )(x.reshape(-1), swf, aux)
    return out_flat.reshape(B, O)


@jax.jit
def kernel(x, grid_points, spline_weights, scale):
    return sc_kan(x, grid_points, spline_weights, scale)


# speedup vs baseline: 2.6140x; 2.6140x over previous
import functools

import jax
import jax.numpy as jnp
from jax import lax
from jax.experimental import pallas as pl
from jax.experimental.pallas import tpu as pltpu, tpu_sc as plsc

_G = 20
_I = 26
_O = 64
_SC_ROWS = 2048
_TC_BLOCK = 2048


def _tc_block_body(x_ref, grid_ref, sw_ref, scale_ref, out_ref, w2_s, gcol_s,
                   *, G, I, O):
    IG = I * G
    g0 = grid_ref[0, 0]
    gL = grid_ref[0, G - 1]

    @pl.when(pl.program_id(0) == 0)
    def _prep():
        w2 = jnp.transpose(sw_ref[...], (0, 2, 1)) * scale_ref[...][:, None, :]
        w2_s[...] = w2.reshape(IG, O)
        cmod = lax.broadcasted_iota(jnp.int32, (8, IG), 1) % G
        gc = jnp.zeros((8, IG), jnp.float32)
        for k in range(G):
            gc = jnp.where(cmod == k, grid_ref[0, k], gc)
        gcol_s[...] = gc

    c_iota = lax.broadcasted_iota(jnp.int32, (2 * I, IG), 1)
    i_iota = lax.broadcasted_iota(jnp.int32, (2 * I, IG), 0) % I
    rep = (c_iota // G == i_iota).astype(jnp.float32)

    xc = jnp.clip(x_ref[...], g0, gL)
    x_hi = (xc.astype(jnp.bfloat16)).astype(jnp.float32)
    x_lo = xc - x_hi
    x2 = jnp.concatenate([x_hi, x_lo], axis=1)
    xrep = lax.dot_general(x2, rep, (((1,), (0,)), ((), ())),
                           preferred_element_type=jnp.float32)

    inv_h = (G - 1) / (gL - g0 + (G - 1) * 1e-08)
    cmat = jnp.maximum(0.0, 1.0 - jnp.abs(xrep - gcol_s[0:1, :]) * inv_h)
    out_ref[...] = lax.dot_general(cmat, w2_s[...], (((1,), (0,)), ((), ())),
                                   preferred_element_type=jnp.float32)


def _tc_kan(x, grid_points, spline_weights, scale, n_rows, out_rows):
    B, I = x.shape
    G = grid_points.shape[0]
    O = spline_weights.shape[1]
    Bb = _TC_BLOCK
    grid2d = grid_points.reshape(1, G)
    body = functools.partial(_tc_block_body, G=G, I=I, O=O)
    return pl.pallas_call(
        body,
        grid=(n_rows // Bb,),
        in_specs=[
            pl.BlockSpec((Bb, I), lambda b: (b, 0)),
            pl.BlockSpec((1, G), lambda b: (0, 0)),
            pl.BlockSpec((I, O, G), lambda b: (0, 0, 0)),
            pl.BlockSpec((I, O), lambda b: (0, 0)),
        ],
        out_specs=pl.BlockSpec((Bb, O), lambda b: (b, 0)),
        out_shape=jax.ShapeDtypeStruct((out_rows, O), jnp.float32),
        scratch_shapes=[
            pltpu.VMEM((I * G, O), jnp.float32),
            pltpu.VMEM((8, I * G), jnp.float32),
        ],
    )(x, grid2d, spline_weights, scale)


def _prep_body(sw_ref, scale_ref, swf_ref):
    swf_ref[...] = sw_ref[...] * scale_ref[...][:, :, None]


def _fold_scale(spline_weights, scale):
    I, O, G = spline_weights.shape
    return pl.pallas_call(
        _prep_body,
        in_specs=[pl.BlockSpec((I, O, G), lambda: (0, 0, 0)),
                  pl.BlockSpec((I, O), lambda: (0, 0))],
        out_specs=pl.BlockSpec((I, O, G), lambda: (0, 0, 0)),
        out_shape=jax.ShapeDtypeStruct((I, O, G), jnp.float32),
    )(spline_weights, scale)


def _sc_body(rows, nchunk, row0, x_hbm, swf_hbm, aux_hbm, out_hbm,
             xs_v, sw_v, aux_v, base_s, t_s, out_v):
    G, I, O = _G, _I, _O
    info = plsc.get_sparse_core_info()
    nc = info.num_cores
    wid = lax.axis_index("s") * nc + lax.axis_index("c")
    base_row = row0 + wid * rows

    pltpu.sync_copy(x_hbm.at[pl.ds(base_row * I, rows * I)], xs_v)
    pltpu.sync_copy(swf_hbm, sw_v)
    pltpu.sync_copy(aux_hbm, aux_v)

    lanes = lax.iota(jnp.int32, 16)
    g0v = aux_v[0:16]
    gLv = aux_v[16:32]
    hv = (gLv - g0v) / (G - 1.0)
    inv_hv = (G - 1.0) / (gLv - g0v)
    invtv = 1.0 / (hv + 1e-08)

    def chunk_body(c, carry):
        row_vec = c * 16 + lanes
        for i in range(I):
            xv = plsc.load_gather(xs_v, [row_vec * I + i])
            xc = jnp.minimum(jnp.maximum(xv, g0v), gLv)
            ji = jnp.clip(((xc - g0v) * inv_hv).astype(jnp.int32), 0, G - 2)
            tv = (xc - (g0v + ji.astype(jnp.float32) * hv)) * invtv
            base_s[i * 16:(i + 1) * 16] = ji + (i * O * G)
            t_s[i * 16:(i + 1) * 16] = tv

        def og_body(og, carry2):
            o20 = og * (16 * G)
            acc = [jnp.zeros((16,), jnp.float32) for _ in range(16)]
            for i in range(I):
                bv = base_s[i * 16:(i + 1) * 16] + o20
                tv = t_s[i * 16:(i + 1) * 16]
                for k in range(16):
                    a0 = bv + (k * G)
                    y0 = plsc.load_gather(sw_v, [a0])
                    y1 = plsc.load_gather(sw_v, [a0 + 1])
                    acc[k] = acc[k] + (y0 + tv * (y1 - y0))
            ob = row_vec * O + og * 16
            for k in range(16):
                plsc.store_scatter(out_v, [ob + k], acc[k])
            return carry2

        lax.fori_loop(0, O // 16, og_body, 0, unroll=False)
        return carry

    lax.fori_loop(0, nchunk, chunk_body, 0, unroll=False)
    pltpu.sync_copy(out_v, out_hbm.at[pl.ds(wid * rows * O, rows * O)])


def _sc_kan(x, grid_points, spline_weights, scale, row0, n_rows):
    B, I = x.shape
    G = grid_points.shape[0]
    O = spline_weights.shape[1]
    info = plsc.get_sparse_core_info()
    nw = info.num_cores * info.num_subcores
    rows = n_rows // nw
    nchunk = rows // 16

    swf = _fold_scale(spline_weights, scale).reshape(-1)
    aux = jnp.concatenate(
        [jnp.full((16,), grid_points[0], jnp.float32),
         jnp.full((16,), grid_points[G - 1], jnp.float32),
         jnp.zeros((96,), jnp.float32)])

    mesh = plsc.VectorSubcoreMesh(core_axis_name="c", subcore_axis_name="s")
    body = functools.partial(_sc_body, rows, nchunk, row0)
    out_flat = pl.kernel(
        body,
        out_type=jax.ShapeDtypeStruct((n_rows * O,), jnp.float32),
        mesh=mesh,
        compiler_params=pltpu.CompilerParams(needs_layout_passes=False),
        scratch_types=[
            pltpu.VMEM((rows * I,), jnp.float32),
            pltpu.VMEM((I * O * G,), jnp.float32),
            pltpu.VMEM((128,), jnp.float32),
            pltpu.VMEM((I * 16,), jnp.int32),
            pltpu.VMEM((I * 16,), jnp.float32),
            pltpu.VMEM((rows * O,), jnp.float32),
        ],
    )(x.reshape(-1), swf, aux)
    return out_flat.reshape(n_rows, O)


@jax.jit
def kernel(x, grid_points, spline_weights, scale):
    B = x.shape[0]
    O = spline_weights.shape[1]
    tc_rows = B - _SC_ROWS
    sc_out = _sc_kan(x, grid_points, spline_weights, scale, tc_rows, _SC_ROWS)
    tc_out = _tc_kan(x, grid_points, spline_weights, scale, tc_rows, B)
    return lax.dynamic_update_slice(tc_out, sc_out, (tc_rows, 0))


# speedup vs baseline: 3.1085x; 1.1892x over previous
import functools

import jax
import jax.numpy as jnp
from jax import lax
from jax.experimental import pallas as pl
from jax.experimental.pallas import tpu as pltpu, tpu_sc as plsc

_G = 20
_I = 26
_O = 64
_SC_ROWS = 512
_TC_BLOCK = 2048


def _tc_block_body(x_ref, grid_ref, sw_ref, scale_ref, out_ref, w2_s, gcol_s,
                   *, G, I, O):
    IG = I * G
    g0 = grid_ref[0, 0]
    gL = grid_ref[0, G - 1]

    @pl.when(pl.program_id(0) == 0)
    def _prep():
        w2 = jnp.transpose(sw_ref[...], (0, 2, 1)) * scale_ref[...][:, None, :]
        w2_s[...] = w2.reshape(IG, O)
        cmod = lax.broadcasted_iota(jnp.int32, (8, IG), 1) % G
        gc = jnp.zeros((8, IG), jnp.float32)
        for k in range(G):
            gc = jnp.where(cmod == k, grid_ref[0, k], gc)
        gcol_s[...] = gc

    c_iota = lax.broadcasted_iota(jnp.int32, (2 * I, IG), 1)
    i_iota = lax.broadcasted_iota(jnp.int32, (2 * I, IG), 0) % I
    rep = (c_iota // G == i_iota).astype(jnp.float32)

    xc = jnp.clip(x_ref[...], g0, gL)
    x_hi = (xc.astype(jnp.bfloat16)).astype(jnp.float32)
    x_lo = xc - x_hi
    x2 = jnp.concatenate([x_hi, x_lo], axis=1)
    xrep = lax.dot_general(x2, rep, (((1,), (0,)), ((), ())),
                           preferred_element_type=jnp.float32)

    inv_h = (G - 1) / (gL - g0 + (G - 1) * 1e-08)
    cmat = jnp.maximum(0.0, 1.0 - jnp.abs(xrep - gcol_s[0:1, :]) * inv_h)
    out_ref[...] = lax.dot_general(cmat, w2_s[...], (((1,), (0,)), ((), ())),
                                   preferred_element_type=jnp.float32)


def _tc_kan(x, grid_points, spline_weights, scale, n_rows, out_rows):
    B, I = x.shape
    G = grid_points.shape[0]
    O = spline_weights.shape[1]
    Bb = _TC_BLOCK
    grid2d = grid_points.reshape(1, G)
    body = functools.partial(_tc_block_body, G=G, I=I, O=O)
    return pl.pallas_call(
        body,
        grid=(n_rows // Bb,),
        in_specs=[
            pl.BlockSpec((Bb, I), lambda b: (b, 0)),
            pl.BlockSpec((1, G), lambda b: (0, 0)),
            pl.BlockSpec((I, O, G), lambda b: (0, 0, 0)),
            pl.BlockSpec((I, O), lambda b: (0, 0)),
        ],
        out_specs=pl.BlockSpec((Bb, O), lambda b: (b, 0)),
        out_shape=jax.ShapeDtypeStruct((out_rows, O), jnp.float32),
        scratch_shapes=[
            pltpu.VMEM((I * G, O), jnp.float32),
            pltpu.VMEM((8, I * G), jnp.float32),
        ],
    )(x, grid2d, spline_weights, scale)


def _prep_body(sw_ref, scale_ref, swf_ref):
    swf_ref[...] = sw_ref[...] * scale_ref[...][:, :, None]


def _fold_scale(spline_weights, scale):
    I, O, G = spline_weights.shape
    return pl.pallas_call(
        _prep_body,
        in_specs=[pl.BlockSpec((I, O, G), lambda: (0, 0, 0)),
                  pl.BlockSpec((I, O), lambda: (0, 0))],
        out_specs=pl.BlockSpec((I, O, G), lambda: (0, 0, 0)),
        out_shape=jax.ShapeDtypeStruct((I, O, G), jnp.float32),
    )(spline_weights, scale)


def _sc_body(rows, nchunk, row0, x_hbm, swf_hbm, aux_hbm, out_hbm,
             xs_v, sw_v, aux_v, base_s, t_s, out_v):
    G, I, O = _G, _I, _O
    info = plsc.get_sparse_core_info()
    nc = info.num_cores
    wid = lax.axis_index("s") * nc + lax.axis_index("c")
    base_row = row0 + wid * rows

    pltpu.sync_copy(x_hbm.at[pl.ds(base_row * I, rows * I)], xs_v)
    pltpu.sync_copy(swf_hbm, sw_v)
    pltpu.sync_copy(aux_hbm, aux_v)

    lanes = lax.iota(jnp.int32, 16)
    g0v = aux_v[0:16]
    gLv = aux_v[16:32]
    hv = (gLv - g0v) / (G - 1.0)
    inv_hv = (G - 1.0) / (gLv - g0v)
    invtv = 1.0 / (hv + 1e-08)

    def chunk_body(c, carry):
        row_vec = c * 16 + lanes
        for i in range(I):
            xv = plsc.load_gather(xs_v, [row_vec * I + i])
            xc = jnp.minimum(jnp.maximum(xv, g0v), gLv)
            ji = jnp.clip(((xc - g0v) * inv_hv).astype(jnp.int32), 0, G - 2)
            tv = (xc - (g0v + ji.astype(jnp.float32) * hv)) * invtv
            base_s[i * 16:(i + 1) * 16] = ji + (i * O * G)
            t_s[i * 16:(i + 1) * 16] = tv

        def og_body(og, carry2):
            o20 = og * (16 * G)
            acc = [jnp.zeros((16,), jnp.float32) for _ in range(16)]
            for i in range(I):
                bv = base_s[i * 16:(i + 1) * 16] + o20
                tv = t_s[i * 16:(i + 1) * 16]
                for k in range(16):
                    a0 = bv + (k * G)
                    y0 = plsc.load_gather(sw_v, [a0])
                    y1 = plsc.load_gather(sw_v, [a0 + 1])
                    acc[k] = acc[k] + (y0 + tv * (y1 - y0))
            ob = row_vec * O + og * 16
            for k in range(16):
                plsc.store_scatter(out_v, [ob + k], acc[k])
            return carry2

        lax.fori_loop(0, O // 16, og_body, 0, unroll=False)
        return carry

    lax.fori_loop(0, nchunk, chunk_body, 0, unroll=False)
    pltpu.sync_copy(out_v, out_hbm.at[pl.ds(wid * rows * O, rows * O)])


def _sc_kan(x, grid_points, spline_weights, scale, row0, n_rows):
    B, I = x.shape
    G = grid_points.shape[0]
    O = spline_weights.shape[1]
    info = plsc.get_sparse_core_info()
    nw = info.num_cores * info.num_subcores
    rows = n_rows // nw
    nchunk = rows // 16

    swf = _fold_scale(spline_weights, scale).reshape(-1)
    aux = jnp.concatenate(
        [jnp.full((16,), grid_points[0], jnp.float32),
         jnp.full((16,), grid_points[G - 1], jnp.float32),
         jnp.zeros((96,), jnp.float32)])

    mesh = plsc.VectorSubcoreMesh(core_axis_name="c", subcore_axis_name="s")
    body = functools.partial(_sc_body, rows, nchunk, row0)
    out_flat = pl.kernel(
        body,
        out_type=jax.ShapeDtypeStruct((n_rows * O,), jnp.float32),
        mesh=mesh,
        compiler_params=pltpu.CompilerParams(needs_layout_passes=False),
        scratch_types=[
            pltpu.VMEM((rows * I,), jnp.float32),
            pltpu.VMEM((I * O * G,), jnp.float32),
            pltpu.VMEM((128,), jnp.float32),
            pltpu.VMEM((I * 16,), jnp.int32),
            pltpu.VMEM((I * 16,), jnp.float32),
            pltpu.VMEM((rows * O,), jnp.float32),
        ],
    )(x.reshape(-1), swf, aux)
    return out_flat.reshape(n_rows, O)


@jax.jit
def kernel(x, grid_points, spline_weights, scale):
    B = x.shape[0]
    O = spline_weights.shape[1]
    tc_rows = B - _SC_ROWS
    sc_out = _sc_kan(x, grid_points, spline_weights, scale, tc_rows, _SC_ROWS)
    tc_out = _tc_kan(x, grid_points, spline_weights, scale, tc_rows, B)
    return lax.dynamic_update_slice(tc_out, sc_out, (tc_rows, 0))
